# baseline (speedup 1.0000x reference)
import functools

import jax
import jax.numpy as jnp
from jax import lax
from jax.experimental import pallas as pl
from jax.experimental.pallas import tpu as pltpu
from jax.experimental.pallas import tpu_sc as plsc

T = 2048
D = 1024
E = 8
K = 2
F = 512
ROUTED_SCALE = 1.0
EPS = 1e-20

LANES = 128
BM = 256
M_PAD = T * K + E * BM
NT = M_PAD // BM
CB = 256
NB = T // CB
NW = 32
TPW = T // NW
DH = D // 2


def _routing_body(x_ref, g_ref, b_ref, pos1_ref, pos2_ref, w1_ref, w2_ref,
                  texp_ref, tact_ref, xpk_ref, c1_scr, c2_scr):
    x = x_ref[...]
    xi = lax.bitcast_convert_type(x, jnp.int32)
    rb = jnp.bitwise_and(lax.shift_right_logical(xi, 16), 1) + 0x7FFF
    xr = xi + rb
    lo = lax.shift_right_logical(xr[:, :DH], 16)
    hi = jnp.bitwise_and(xr[:, DH:], jnp.int32(-65536))
    xpk_ref[...] = jnp.bitwise_or(lo, hi)
    logits = lax.dot_general(x, g_ref[...], (((1,), (1,)), ((), ())),
                             preferred_element_type=jnp.float32)
    scores = jax.nn.sigmoid(logits)
    lane = lax.broadcasted_iota(jnp.int32, (T, LANES), 1)
    choice = scores + b_ref[...]
    m1 = jnp.max(choice, axis=1, keepdims=True)
    i1 = jnp.min(jnp.where(choice >= m1, lane, LANES), axis=1, keepdims=True)
    sel1 = lane == i1
    w1 = jnp.sum(jnp.where(sel1, scores, 0.0), axis=1, keepdims=True)
    choice2 = jnp.where(sel1, -1e30, choice)
    m2 = jnp.max(choice2, axis=1, keepdims=True)
    i2 = jnp.min(jnp.where(choice2 >= m2, lane, LANES), axis=1, keepdims=True)
    sel2 = lane == i2
    w2 = jnp.sum(jnp.where(sel2, scores, 0.0), axis=1, keepdims=True)
    den = w1 + w2 + EPS
    w1 = w1 / den * ROUTED_SCALE
    w2 = w2 / den * ROUTED_SCALE
    w1_ref[...] = jnp.broadcast_to(w1, (T, LANES))
    w2_ref[...] = jnp.broadcast_to(w2, (T, LANES))

    oh1 = sel1.astype(jnp.float32)
    oh2 = sel2.astype(jnp.float32)
    c1_scr[...] = oh1
    c2_scr[...] = oh2
    ltexc = (lax.broadcasted_iota(jnp.int32, (CB, CB), 0)
             > lax.broadcasted_iota(jnp.int32, (CB, CB), 1)).astype(jnp.float32)

    def blk(b, carry):
        car1, car2 = carry
        s = pl.ds(b * CB, CB)
        blk1 = c1_scr[s, :]
        blk2 = c2_scr[s, :]
        c1_scr[s, :] = lax.dot_general(
            ltexc, blk1, (((1,), (0,)), ((), ())),
            preferred_element_type=jnp.float32) + car1
        c2_scr[s, :] = lax.dot_general(
            ltexc, blk2, (((1,), (0,)), ((), ())),
            preferred_element_type=jnp.float32) + car2
        return (car1 + jnp.sum(blk1, axis=0, keepdims=True),
                car2 + jnp.sum(blk2, axis=0, keepdims=True))

    zero = jnp.zeros((1, LANES), jnp.float32)
    tot1, tot2 = lax.fori_loop(0, NB, blk, (zero, zero))

    counts = (tot1 + tot2).astype(jnp.int32)
    padded = lax.shift_left(lax.shift_right_logical(counts + (BM - 1), 8), 8)
    lte = (lax.broadcasted_iota(jnp.int32, (LANES, LANES), 0)
           < lax.broadcasted_iota(jnp.int32, (LANES, LANES), 1)).astype(jnp.float32)
    seg = lax.dot_general(padded.astype(jnp.float32), lte,
                          (((1,), (0,)), ((), ())),
                          preferred_element_type=jnp.float32)
    c1v = c1_scr[...]
    c2v = c2_scr[...]
    pos1 = jnp.sum((seg + c1v) * oh1, axis=1, keepdims=True)
    pos2 = jnp.sum((seg + tot1 + c2v) * oh2, axis=1, keepdims=True)
    pos1_ref[...] = jnp.broadcast_to(pos1.astype(jnp.int32), (T, LANES))
    pos2_ref[...] = jnp.broadcast_to(pos2.astype(jnp.int32), (T, LANES))

    segi = seg.astype(jnp.int32)
    tstart = lax.broadcasted_iota(jnp.int32, (32, LANES), 0) * BM
    act2 = (segi <= tstart) & (tstart < segi + counts)
    lane2 = lax.broadcasted_iota(jnp.int32, (32, LANES), 1)
    tact = jnp.sum(act2.astype(jnp.int32), axis=1, keepdims=True)
    texp = jnp.where(tact == 1,
                     jnp.sum(jnp.where(act2, lane2, 0), axis=1, keepdims=True),
                     E - 1)
    texp_ref[...] = jnp.broadcast_to(texp, (32, LANES))
    tact_ref[...] = jnp.broadcast_to(tact, (32, LANES))


def _routing_call(x, gate_pad, bias_pad):
    return pl.pallas_call(
        _routing_body,
        out_shape=(
            jax.ShapeDtypeStruct((T, LANES), jnp.int32),
            jax.ShapeDtypeStruct((T, LANES), jnp.int32),
            jax.ShapeDtypeStruct((T, LANES), jnp.float32),
            jax.ShapeDtypeStruct((T, LANES), jnp.float32),
            jax.ShapeDtypeStruct((32, LANES), jnp.int32),
            jax.ShapeDtypeStruct((32, LANES), jnp.int32),
            jax.ShapeDtypeStruct((T, DH), jnp.int32),
        ),
        scratch_shapes=[
            pltpu.VMEM((T, LANES), jnp.float32),
            pltpu.VMEM((T, LANES), jnp.float32),
        ],
    )(x, gate_pad, bias_pad)


def _gmm_body(texp_ref, tact_ref, xs_ref, wg_ref, wu_ref, wd_ref, ws_ref, y_ref,
              wgbf_scr, wubf_scr, wdbf_scr):
    i = pl.program_id(0)
    changed = jnp.logical_or(
        i == 0, texp_ref[i] != texp_ref[jnp.maximum(i - 1, 0)])

    @pl.when(jnp.logical_and(tact_ref[i] == 1, changed))
    def _():
        wgbf_scr[...] = wg_ref[0].astype(jnp.bfloat16)
        wubf_scr[...] = wu_ref[0].astype(jnp.bfloat16)
        wdbf_scr[...] = wd_ref[0].astype(jnp.bfloat16)

    @pl.when(tact_ref[i] == 1)
    def _():
        pk = xs_ref[...]
        xlo = lax.bitcast_convert_type(
            lax.shift_left(pk, 16), jnp.float32).astype(jnp.bfloat16)
        xhi = lax.bitcast_convert_type(
            jnp.bitwise_and(pk, jnp.int32(-65536)), jnp.float32).astype(jnp.bfloat16)
        xt = jnp.concatenate([xlo, xhi], axis=1)
        g = jnp.dot(xt, wgbf_scr[...], preferred_element_type=jnp.float32)
        u = jnp.dot(xt, wubf_scr[...], preferred_element_type=jnp.float32)
        act = g * jax.nn.sigmoid(g) * u
        act = act * ws_ref[:, :1]
        y_ref[...] = jnp.dot(act.astype(jnp.bfloat16), wdbf_scr[...],
                             preferred_element_type=jnp.float32)


def _gmm_call(texp, tact, xs, w_gate, w_up, w_down, wsorted):
    grid_spec = pltpu.PrefetchScalarGridSpec(
        num_scalar_prefetch=2,
        grid=(NT,),
        in_specs=[
            pl.BlockSpec((BM, DH), lambda i, texp, tact: (i, 0)),
            pl.BlockSpec((1, D, F), lambda i, texp, tact: (texp[i], 0, 0)),
            pl.BlockSpec((1, D, F), lambda i, texp, tact: (texp[i], 0, 0)),
            pl.BlockSpec((1, F, D), lambda i, texp, tact: (texp[i], 0, 0)),
            pl.BlockSpec((BM, LANES), lambda i, texp, tact: (i, 0)),
        ],
        out_specs=pl.BlockSpec((BM, D), lambda i, texp, tact: (i, 0)),
        scratch_shapes=[
            pltpu.VMEM((D, F), jnp.bfloat16),
            pltpu.VMEM((D, F), jnp.bfloat16),
            pltpu.VMEM((F, D), jnp.bfloat16),
        ],
    )
    return pl.pallas_call(
        _gmm_body,
        grid_spec=grid_spec,
        out_shape=jax.ShapeDtypeStruct((M_PAD, D), jnp.float32),
    )(texp, tact, xs, w_gate, w_up, w_down, wsorted)


def _dispatch_body(x_hbm, pos_hbm, w1_hbm, w2_hbm, xs_hbm, ws_hbm,
                   xbuf, w1buf, w2buf, idx0, idx1, sem):
    w = lax.axis_index("s") * 2 + lax.axis_index("c")
    base = w * TPW
    c0 = pltpu.async_copy(x_hbm.at[pl.ds(base, TPW)], xbuf, sem)
    c1 = pltpu.async_copy(pos_hbm.at[w, 0], idx0, sem)
    c2 = pltpu.async_copy(pos_hbm.at[w, 1], idx1, sem)
    c3 = pltpu.async_copy(w1_hbm.at[pl.ds(base, TPW)], w1buf, sem)
    c4 = pltpu.async_copy(w2_hbm.at[pl.ds(base, TPW)], w2buf, sem)
    c0.wait(); c1.wait(); c2.wait(); c3.wait(); c4.wait()
    s0 = pltpu.async_copy(xbuf, xs_hbm.at[idx0], sem)
    s1 = pltpu.async_copy(xbuf, xs_hbm.at[idx1], sem)
    s2 = pltpu.async_copy(w1buf, ws_hbm.at[idx0], sem)
    s3 = pltpu.async_copy(w2buf, ws_hbm.at[idx1], sem)
    s0.wait(); s1.wait(); s2.wait(); s3.wait()


@functools.cache
def _dispatch_kernel():
    mesh = plsc.VectorSubcoreMesh(core_axis_name="c", subcore_axis_name="s")
    return pl.kernel(
        _dispatch_body,
        out_type=(
            jax.ShapeDtypeStruct((M_PAD, DH), jnp.int32),
            jax.ShapeDtypeStruct((M_PAD, LANES), jnp.float32),
        ),
        mesh=mesh,
        scratch_types=[
            pltpu.VMEM((TPW, DH), jnp.int32),
            pltpu.VMEM((TPW, LANES), jnp.float32),
            pltpu.VMEM((TPW, LANES), jnp.float32),
            pltpu.VMEM((TPW,), jnp.int32),
            pltpu.VMEM((TPW,), jnp.int32),
            pltpu.SemaphoreType.DMA,
        ],
    )


HT = TPW // 2


def _combine_body(y_hbm, pos_hbm, out_hbm, ybuf0, ybuf1, idx0, idx1, sem):
    w = lax.axis_index("s") * 2 + lax.axis_index("c")
    base = w * TPW
    c1 = pltpu.async_copy(pos_hbm.at[w, 0], idx0, sem)
    c2 = pltpu.async_copy(pos_hbm.at[w, 1], idx1, sem)
    c1.wait(); c2.wait()
    for h in range(2):
        g0 = pltpu.async_copy(y_hbm.at[idx0.at[pl.ds(h * HT, HT)]], ybuf0, sem)
        g1 = pltpu.async_copy(y_hbm.at[idx1.at[pl.ds(h * HT, HT)]], ybuf1, sem)
        g0.wait(); g1.wait()

        def row(i, _):
            for cpart in range(D // 16):
                s = pl.ds(cpart * 16, 16)
                ybuf0[i, s] = ybuf0[i, s] + ybuf1[i, s]
            return 0

        lax.fori_loop(0, HT, row, 0)
        pltpu.sync_copy(ybuf0, out_hbm.at[pl.ds(base + h * HT, HT)])


@functools.cache
def _combine_kernel():
    mesh = plsc.VectorSubcoreMesh(core_axis_name="c", subcore_axis_name="s")
    return pl.kernel(
        _combine_body,
        out_type=jax.ShapeDtypeStruct((T, D), jnp.float32),
        mesh=mesh,
        scratch_types=[
            pltpu.VMEM((HT, D), jnp.float32),
            pltpu.VMEM((HT, D), jnp.float32),
            pltpu.VMEM((TPW,), jnp.int32),
            pltpu.VMEM((TPW,), jnp.int32),
            pltpu.SemaphoreType.DMA,
        ],
    )


def kernel(hidden_states, gate_w, router_bias, w_gate, w_up, w_down):
    x = hidden_states.astype(jnp.float32)
    gate_pad = jnp.zeros((LANES, D), jnp.float32).at[:E].set(gate_w)
    bias_pad = jnp.full((1, LANES), -1e30, jnp.float32).at[0, :E].set(router_bias)

    pos1b, pos2b, w1b, w2b, texp_b, tact_b, x_pk = _routing_call(x, gate_pad, bias_pad)
    texp = texp_b[:NT, 0]
    tact = tact_b[:NT, 0]
    pos_sc = jnp.stack(
        [pos1b[:, 0].reshape(NW, TPW), pos2b[:, 0].reshape(NW, TPW)], axis=1)

    xs, wsorted = _dispatch_kernel()(x_pk, pos_sc, w1b, w2b)
    y = _gmm_call(texp, tact, xs, w_gate, w_up, w_down, wsorted)
    return _combine_kernel()(y, pos_sc)


# speedup vs baseline: 1.0474x; 1.0474x over previous
import functools

import jax
import jax.numpy as jnp
from jax import lax
from jax.experimental import pallas as pl
from jax.experimental.pallas import tpu as pltpu
from jax.experimental.pallas import tpu_sc as plsc

T = 2048
D = 1024
E = 8
K = 2
F = 512
ROUTED_SCALE = 1.0
EPS = 1e-20

LANES = 128
BM = 256
M_PAD = T * K + E * BM
NT = M_PAD // BM
CB = 256
NB = T // CB
NW = 32
TPW = T // NW
DH = D // 2


def _routing_body(x_ref, g_ref, b_ref, pos1_ref, pos2_ref, w1_ref, w2_ref,
                  texp_ref, tact_ref, xpk_ref, c1_scr, c2_scr):
    x = x_ref[...]
    xi = lax.bitcast_convert_type(x, jnp.int32)
    rb = jnp.bitwise_and(lax.shift_right_logical(xi, 16), 1) + 0x7FFF
    xr = xi + rb
    lo = lax.shift_right_logical(xr[:, :DH], 16)
    hi = jnp.bitwise_and(xr[:, DH:], jnp.int32(-65536))
    xpk_ref[...] = jnp.bitwise_or(lo, hi)
    logits = lax.dot_general(x, g_ref[...], (((1,), (1,)), ((), ())),
                             preferred_element_type=jnp.float32)
    scores = jax.nn.sigmoid(logits)
    lane = lax.broadcasted_iota(jnp.int32, (T, LANES), 1)
    choice = scores + b_ref[...]
    m1 = jnp.max(choice, axis=1, keepdims=True)
    i1 = jnp.min(jnp.where(choice >= m1, lane, LANES), axis=1, keepdims=True)
    sel1 = lane == i1
    w1 = jnp.sum(jnp.where(sel1, scores, 0.0), axis=1, keepdims=True)
    choice2 = jnp.where(sel1, -1e30, choice)
    m2 = jnp.max(choice2, axis=1, keepdims=True)
    i2 = jnp.min(jnp.where(choice2 >= m2, lane, LANES), axis=1, keepdims=True)
    sel2 = lane == i2
    w2 = jnp.sum(jnp.where(sel2, scores, 0.0), axis=1, keepdims=True)
    den = w1 + w2 + EPS
    w1 = w1 / den * ROUTED_SCALE
    w2 = w2 / den * ROUTED_SCALE
    w1_ref[...] = jnp.broadcast_to(w1, (T, LANES))
    w2_ref[...] = jnp.broadcast_to(w2, (T, LANES))

    oh1 = sel1.astype(jnp.float32)
    oh2 = sel2.astype(jnp.float32)
    c1_scr[...] = oh1
    c2_scr[...] = oh2
    ltexc = (lax.broadcasted_iota(jnp.int32, (CB, CB), 0)
             > lax.broadcasted_iota(jnp.int32, (CB, CB), 1)).astype(jnp.float32)

    def blk(b, carry):
        car1, car2 = carry
        s = pl.ds(b * CB, CB)
        blk1 = c1_scr[s, :]
        blk2 = c2_scr[s, :]
        c1_scr[s, :] = lax.dot_general(
            ltexc, blk1, (((1,), (0,)), ((), ())),
            preferred_element_type=jnp.float32) + car1
        c2_scr[s, :] = lax.dot_general(
            ltexc, blk2, (((1,), (0,)), ((), ())),
            preferred_element_type=jnp.float32) + car2
        return (car1 + jnp.sum(blk1, axis=0, keepdims=True),
                car2 + jnp.sum(blk2, axis=0, keepdims=True))

    zero = jnp.zeros((1, LANES), jnp.float32)
    tot1, tot2 = lax.fori_loop(0, NB, blk, (zero, zero))

    counts = (tot1 + tot2).astype(jnp.int32)
    padded = lax.shift_left(lax.shift_right_logical(counts + (BM - 1), 8), 8)
    lte = (lax.broadcasted_iota(jnp.int32, (LANES, LANES), 0)
           < lax.broadcasted_iota(jnp.int32, (LANES, LANES), 1)).astype(jnp.float32)
    seg = lax.dot_general(padded.astype(jnp.float32), lte,
                          (((1,), (0,)), ((), ())),
                          preferred_element_type=jnp.float32)
    c1v = c1_scr[...]
    c2v = c2_scr[...]
    pos1 = jnp.sum((seg + c1v) * oh1, axis=1, keepdims=True)
    pos2 = jnp.sum((seg + tot1 + c2v) * oh2, axis=1, keepdims=True)
    pos1_ref[...] = jnp.broadcast_to(pos1.astype(jnp.int32), (T, LANES))
    pos2_ref[...] = jnp.broadcast_to(pos2.astype(jnp.int32), (T, LANES))

    segi = seg.astype(jnp.int32)
    tstart = lax.broadcasted_iota(jnp.int32, (32, LANES), 0) * BM
    act2 = (segi <= tstart) & (tstart < segi + counts)
    lane2 = lax.broadcasted_iota(jnp.int32, (32, LANES), 1)
    tact = jnp.sum(act2.astype(jnp.int32), axis=1, keepdims=True)
    texp = jnp.where(tact == 1,
                     jnp.sum(jnp.where(act2, lane2, 0), axis=1, keepdims=True),
                     E - 1)
    texp_ref[...] = jnp.broadcast_to(texp, (32, LANES))
    tact_ref[...] = jnp.broadcast_to(tact, (32, LANES))


def _routing_call(x, gate_pad, bias_pad):
    return pl.pallas_call(
        _routing_body,
        out_shape=(
            jax.ShapeDtypeStruct((T, LANES), jnp.int32),
            jax.ShapeDtypeStruct((T, LANES), jnp.int32),
            jax.ShapeDtypeStruct((T, LANES), jnp.float32),
            jax.ShapeDtypeStruct((T, LANES), jnp.float32),
            jax.ShapeDtypeStruct((32, LANES), jnp.int32),
            jax.ShapeDtypeStruct((32, LANES), jnp.int32),
            jax.ShapeDtypeStruct((T, DH), jnp.int32),
        ),
        scratch_shapes=[
            pltpu.VMEM((T, LANES), jnp.float32),
            pltpu.VMEM((T, LANES), jnp.float32),
        ],
    )(x, gate_pad, bias_pad)


def _gmm_body(texp_ref, tact_ref, xs_ref, wg_ref, wu_ref, wd_ref, ws_ref, y_ref,
              wgbf_scr, wubf_scr, wdbf_scr):
    i = pl.program_id(0)
    changed = jnp.logical_or(
        i == 0, texp_ref[i] != texp_ref[jnp.maximum(i - 1, 0)])

    @pl.when(jnp.logical_and(tact_ref[i] == 1, changed))
    def _():
        wgbf_scr[...] = wg_ref[0].astype(jnp.bfloat16)
        wubf_scr[...] = wu_ref[0].astype(jnp.bfloat16)
        wdbf_scr[...] = wd_ref[0].astype(jnp.bfloat16)

    @pl.when(tact_ref[i] == 1)
    def _():
        pk = xs_ref[...]
        xlo = lax.bitcast_convert_type(
            lax.shift_left(pk, 16), jnp.float32).astype(jnp.bfloat16)
        xhi = lax.bitcast_convert_type(
            jnp.bitwise_and(pk, jnp.int32(-65536)), jnp.float32).astype(jnp.bfloat16)
        xt = jnp.concatenate([xlo, xhi], axis=1)
        g = jnp.dot(xt, wgbf_scr[...], preferred_element_type=jnp.float32)
        u = jnp.dot(xt, wubf_scr[...], preferred_element_type=jnp.float32)
        act = g * jax.nn.sigmoid(g) * u
        act = act * ws_ref[:, :1]
        yv = jnp.dot(act.astype(jnp.bfloat16), wdbf_scr[...],
                     preferred_element_type=jnp.float32)
        yi = lax.bitcast_convert_type(yv, jnp.int32)
        rb = jnp.bitwise_and(lax.shift_right_logical(yi, 16), 1) + 0x7FFF
        yr = yi + rb
        y_ref[...] = jnp.bitwise_or(
            lax.shift_right_logical(yr[:, :DH], 16),
            jnp.bitwise_and(yr[:, DH:], jnp.int32(-65536)))


def _gmm_call(texp, tact, xs, w_gate, w_up, w_down, wsorted):
    grid_spec = pltpu.PrefetchScalarGridSpec(
        num_scalar_prefetch=2,
        grid=(NT,),
        in_specs=[
            pl.BlockSpec((BM, DH), lambda i, texp, tact: (i, 0)),
            pl.BlockSpec((1, D, F), lambda i, texp, tact: (texp[i], 0, 0)),
            pl.BlockSpec((1, D, F), lambda i, texp, tact: (texp[i], 0, 0)),
            pl.BlockSpec((1, F, D), lambda i, texp, tact: (texp[i], 0, 0)),
            pl.BlockSpec((BM, LANES), lambda i, texp, tact: (i, 0)),
        ],
        out_specs=pl.BlockSpec((BM, DH), lambda i, texp, tact: (i, 0)),
        scratch_shapes=[
            pltpu.VMEM((D, F), jnp.bfloat16),
            pltpu.VMEM((D, F), jnp.bfloat16),
            pltpu.VMEM((F, D), jnp.bfloat16),
        ],
    )
    return pl.pallas_call(
        _gmm_body,
        grid_spec=grid_spec,
        out_shape=jax.ShapeDtypeStruct((M_PAD, DH), jnp.int32),
    )(texp, tact, xs, w_gate, w_up, w_down, wsorted)


def _dispatch_body(x_hbm, pos_hbm, w1_hbm, w2_hbm, xs_hbm, ws_hbm,
                   xbuf, w1buf, w2buf, idx0, idx1, sem):
    w = lax.axis_index("s") * 2 + lax.axis_index("c")
    base = w * TPW
    c0 = pltpu.async_copy(x_hbm.at[pl.ds(base, TPW)], xbuf, sem)
    c1 = pltpu.async_copy(pos_hbm.at[w, 0], idx0, sem)
    c2 = pltpu.async_copy(pos_hbm.at[w, 1], idx1, sem)
    c3 = pltpu.async_copy(w1_hbm.at[pl.ds(base, TPW)], w1buf, sem)
    c4 = pltpu.async_copy(w2_hbm.at[pl.ds(base, TPW)], w2buf, sem)
    c0.wait(); c1.wait(); c2.wait(); c3.wait(); c4.wait()
    s0 = pltpu.async_copy(xbuf, xs_hbm.at[idx0], sem)
    s1 = pltpu.async_copy(xbuf, xs_hbm.at[idx1], sem)
    s2 = pltpu.async_copy(w1buf, ws_hbm.at[idx0], sem)
    s3 = pltpu.async_copy(w2buf, ws_hbm.at[idx1], sem)
    s0.wait(); s1.wait(); s2.wait(); s3.wait()


@functools.cache
def _dispatch_kernel():
    mesh = plsc.VectorSubcoreMesh(core_axis_name="c", subcore_axis_name="s")
    return pl.kernel(
        _dispatch_body,
        out_type=(
            jax.ShapeDtypeStruct((M_PAD, DH), jnp.int32),
            jax.ShapeDtypeStruct((M_PAD, LANES), jnp.float32),
        ),
        mesh=mesh,
        scratch_types=[
            pltpu.VMEM((TPW, DH), jnp.int32),
            pltpu.VMEM((TPW, LANES), jnp.float32),
            pltpu.VMEM((TPW, LANES), jnp.float32),
            pltpu.VMEM((TPW,), jnp.int32),
            pltpu.VMEM((TPW,), jnp.int32),
            pltpu.SemaphoreType.DMA,
        ],
    )


HT = TPW // 2


def _combine_body(y_hbm, pos_hbm, y0_hbm, y1_hbm, ybuf0, ybuf1, idx0, idx1, sem):
    w = lax.axis_index("s") * 2 + lax.axis_index("c")
    base = w * TPW
    c1 = pltpu.async_copy(pos_hbm.at[w, 0], idx0, sem)
    c2 = pltpu.async_copy(pos_hbm.at[w, 1], idx1, sem)
    c1.wait(); c2.wait()
    g0 = pltpu.async_copy(y_hbm.at[idx0], ybuf0, sem)
    g1 = pltpu.async_copy(y_hbm.at[idx1], ybuf1, sem)
    g0.wait(); g1.wait()
    s0 = pltpu.async_copy(ybuf0, y0_hbm.at[pl.ds(base, TPW)], sem)
    s1 = pltpu.async_copy(ybuf1, y1_hbm.at[pl.ds(base, TPW)], sem)
    s0.wait(); s1.wait()


@functools.cache
def _combine_kernel():
    mesh = plsc.VectorSubcoreMesh(core_axis_name="c", subcore_axis_name="s")
    return pl.kernel(
        _combine_body,
        out_type=(
            jax.ShapeDtypeStruct((T, DH), jnp.int32),
            jax.ShapeDtypeStruct((T, DH), jnp.int32),
        ),
        mesh=mesh,
        scratch_types=[
            pltpu.VMEM((TPW, DH), jnp.int32),
            pltpu.VMEM((TPW, DH), jnp.int32),
            pltpu.VMEM((TPW,), jnp.int32),
            pltpu.VMEM((TPW,), jnp.int32),
            pltpu.SemaphoreType.DMA,
        ],
    )


def _final_body(y0_ref, y1_ref, o_ref):
    pk0 = y0_ref[...]
    pk1 = y1_ref[...]
    lo = (lax.bitcast_convert_type(lax.shift_left(pk0, 16), jnp.float32)
          + lax.bitcast_convert_type(lax.shift_left(pk1, 16), jnp.float32))
    hi = (lax.bitcast_convert_type(
              jnp.bitwise_and(pk0, jnp.int32(-65536)), jnp.float32)
          + lax.bitcast_convert_type(
              jnp.bitwise_and(pk1, jnp.int32(-65536)), jnp.float32))
    o_ref[...] = jnp.concatenate([lo, hi], axis=1)


def _final_call(y0, y1):
    return pl.pallas_call(
        _final_body,
        grid=(T // BM,),
        in_specs=[
            pl.BlockSpec((BM, DH), lambda i: (i, 0)),
            pl.BlockSpec((BM, DH), lambda i: (i, 0)),
        ],
        out_specs=pl.BlockSpec((BM, D), lambda i: (i, 0)),
        out_shape=jax.ShapeDtypeStruct((T, D), jnp.float32),
    )(y0, y1)


def kernel(hidden_states, gate_w, router_bias, w_gate, w_up, w_down):
    x = hidden_states.astype(jnp.float32)
    gate_pad = jnp.zeros((LANES, D), jnp.float32).at[:E].set(gate_w)
    bias_pad = jnp.full((1, LANES), -1e30, jnp.float32).at[0, :E].set(router_bias)

    pos1b, pos2b, w1b, w2b, texp_b, tact_b, x_pk = _routing_call(x, gate_pad, bias_pad)
    texp = texp_b[:NT, 0]
    tact = tact_b[:NT, 0]
    pos_sc = jnp.stack(
        [pos1b[:, 0].reshape(NW, TPW), pos2b[:, 0].reshape(NW, TPW)], axis=1)

    xs, wsorted = _dispatch_kernel()(x_pk, pos_sc, w1b, w2b)
    y = _gmm_call(texp, tact, xs, w_gate, w_up, w_down, wsorted)
    y0, y1 = _combine_kernel()(y, pos_sc)
    return _final_call(y0, y1)


# speedup vs baseline: 1.1490x; 1.0971x over previous
import functools

import jax
import jax.numpy as jnp
from jax import lax
from jax.experimental import pallas as pl
from jax.experimental.pallas import tpu as pltpu
from jax.experimental.pallas import tpu_sc as plsc

T = 2048
D = 1024
E = 8
K = 2
F = 512
ROUTED_SCALE = 1.0
EPS = 1e-20

LANES = 128
BM = 256
M_PAD = T * K + E * BM
NT = M_PAD // BM
CB = 256
NB = T // CB
NW = 32
TPW = T // NW
DH = D // 2


def _routing_body(x_ref, g_ref, b_ref, posw_ref, w1_ref, w2_ref,
                  texp_ref, tact_ref, xpk_ref, c1_scr, c2_scr):
    x = x_ref[...]
    xi = lax.bitcast_convert_type(x, jnp.int32)
    rb = jnp.bitwise_and(lax.shift_right_logical(xi, 16), 1) + 0x7FFF
    xr = xi + rb
    lo = lax.shift_right_logical(xr[:, :DH], 16)
    hi = jnp.bitwise_and(xr[:, DH:], jnp.int32(-65536))
    xpk_ref[...] = jnp.bitwise_or(lo, hi)
    logits8 = lax.dot_general(x, g_ref[...], (((1,), (1,)), ((), ())),
                              preferred_element_type=jnp.float32)
    scores8 = jax.nn.sigmoid(logits8)
    lane = lax.broadcasted_iota(jnp.int32, (T, LANES), 1)
    scores = jnp.concatenate(
        [scores8, jnp.zeros((T, LANES - E), jnp.float32)], axis=1)
    choice = jnp.concatenate(
        [scores8 + b_ref[...], jnp.full((T, LANES - E), -1e30, jnp.float32)],
        axis=1)
    m1 = jnp.max(choice, axis=1, keepdims=True)
    i1 = jnp.min(jnp.where(choice >= m1, lane, LANES), axis=1, keepdims=True)
    sel1 = lane == i1
    w1 = jnp.sum(jnp.where(sel1, scores, 0.0), axis=1, keepdims=True)
    choice2 = jnp.where(sel1, -1e30, choice)
    m2 = jnp.max(choice2, axis=1, keepdims=True)
    i2 = jnp.min(jnp.where(choice2 >= m2, lane, LANES), axis=1, keepdims=True)
    sel2 = lane == i2
    w2 = jnp.sum(jnp.where(sel2, scores, 0.0), axis=1, keepdims=True)
    den = w1 + w2 + EPS
    w1 = w1 / den * ROUTED_SCALE
    w2 = w2 / den * ROUTED_SCALE
    w1_ref[...] = jnp.broadcast_to(w1, (T, LANES))
    w2_ref[...] = jnp.broadcast_to(w2, (T, LANES))

    oh1 = sel1.astype(jnp.float32)
    oh2 = sel2.astype(jnp.float32)
    c1_scr[...] = oh1
    c2_scr[...] = oh2
    ltexc = (lax.broadcasted_iota(jnp.int32, (CB, CB), 0)
             > lax.broadcasted_iota(jnp.int32, (CB, CB), 1)).astype(jnp.float32)

    def blk(b, carry):
        car1, car2 = carry
        s = pl.ds(b * CB, CB)
        blk1 = c1_scr[s, :]
        blk2 = c2_scr[s, :]
        c1_scr[s, :] = lax.dot_general(
            ltexc, blk1, (((1,), (0,)), ((), ())),
            preferred_element_type=jnp.float32) + car1
        c2_scr[s, :] = lax.dot_general(
            ltexc, blk2, (((1,), (0,)), ((), ())),
            preferred_element_type=jnp.float32) + car2
        return (car1 + jnp.sum(blk1, axis=0, keepdims=True),
                car2 + jnp.sum(blk2, axis=0, keepdims=True))

    zero = jnp.zeros((1, LANES), jnp.float32)
    tot1, tot2 = lax.fori_loop(0, NB, blk, (zero, zero))

    counts = (tot1 + tot2).astype(jnp.int32)
    padded = lax.shift_left(lax.shift_right_logical(counts + (BM - 1), 8), 8)
    lte = (lax.broadcasted_iota(jnp.int32, (LANES, LANES), 0)
           < lax.broadcasted_iota(jnp.int32, (LANES, LANES), 1)).astype(jnp.float32)
    seg = lax.dot_general(padded.astype(jnp.float32), lte,
                          (((1,), (0,)), ((), ())),
                          preferred_element_type=jnp.float32)
    c1v = c1_scr[...]
    c2v = c2_scr[...]
    pos1 = jnp.sum((seg + c1v) * oh1, axis=1, keepdims=True)
    pos2 = jnp.sum((seg + tot1 + c2v) * oh2, axis=1, keepdims=True)
    tok0 = lax.broadcasted_iota(jnp.int32, (T, LANES), 0)
    eqj = (lane == jnp.bitwise_and(tok0, TPW - 1)).astype(jnp.float32)
    wsel = (lax.broadcasted_iota(jnp.int32, (NW, T), 0)
            == lax.shift_right_logical(
                lax.broadcasted_iota(jnp.int32, (NW, T), 1), 6)
            ).astype(jnp.float32)
    cdims = (((1,), (0,)), ((), ()))

    def relayout(posv):
        p_hi = jnp.floor(posv * (1.0 / 32.0))
        p_lo = posv - 32.0 * p_hi
        hi = lax.dot_general(wsel, p_hi * eqj, cdims,
                             preferred_element_type=jnp.float32)
        lo = lax.dot_general(wsel, p_lo * eqj, cdims,
                             preferred_element_type=jnp.float32)
        return 32.0 * hi + lo

    posw_ref[...] = jnp.concatenate(
        [relayout(pos1), relayout(pos2)], axis=0).astype(jnp.int32)

    segi = seg.astype(jnp.int32)
    tstart = lax.broadcasted_iota(jnp.int32, (32, LANES), 0) * BM
    act2 = (segi <= tstart) & (tstart < segi + counts)
    lane2 = lax.broadcasted_iota(jnp.int32, (32, LANES), 1)
    tact = jnp.sum(act2.astype(jnp.int32), axis=1, keepdims=True)
    texp = jnp.where(tact == 1,
                     jnp.sum(jnp.where(act2, lane2, 0), axis=1, keepdims=True),
                     E - 1)
    texp_ref[...] = jnp.broadcast_to(texp, (32, LANES))
    tact_ref[...] = jnp.broadcast_to(tact, (32, LANES))


def _routing_call(x, gate_w, bias_2d):
    return pl.pallas_call(
        _routing_body,
        out_shape=(
            jax.ShapeDtypeStruct((2 * NW, LANES), jnp.int32),
            jax.ShapeDtypeStruct((T, LANES), jnp.float32),
            jax.ShapeDtypeStruct((T, LANES), jnp.float32),
            jax.ShapeDtypeStruct((32, LANES), jnp.int32),
            jax.ShapeDtypeStruct((32, LANES), jnp.int32),
            jax.ShapeDtypeStruct((T, DH), jnp.int32),
        ),
        scratch_shapes=[
            pltpu.VMEM((T, LANES), jnp.float32),
            pltpu.VMEM((T, LANES), jnp.float32),
        ],
    )(x, gate_w, bias_2d)


def _gmm_body(texp_ref, tact_ref, xs_ref, wg_ref, wu_ref, wd_ref, ws_ref, y_ref,
              wgbf_scr, wubf_scr, wdbf_scr):
    i = pl.program_id(0)
    changed = jnp.logical_or(
        i == 0, texp_ref[i] != texp_ref[jnp.maximum(i - 1, 0)])

    @pl.when(jnp.logical_and(tact_ref[i] == 1, changed))
    def _():
        wgbf_scr[...] = wg_ref[0].astype(jnp.bfloat16)
        wubf_scr[...] = wu_ref[0].astype(jnp.bfloat16)
        wdbf_scr[...] = wd_ref[0].astype(jnp.bfloat16)

    @pl.when(tact_ref[i] == 1)
    def _():
        pk = xs_ref[...]
        xlo = lax.bitcast_convert_type(
            lax.shift_left(pk, 16), jnp.float32).astype(jnp.bfloat16)
        xhi = lax.bitcast_convert_type(
            jnp.bitwise_and(pk, jnp.int32(-65536)), jnp.float32).astype(jnp.bfloat16)
        xt = jnp.concatenate([xlo, xhi], axis=1)
        g = jnp.dot(xt, wgbf_scr[...], preferred_element_type=jnp.float32)
        u = jnp.dot(xt, wubf_scr[...], preferred_element_type=jnp.float32)
        act = g * jax.nn.sigmoid(g) * u
        act = act * ws_ref[:, :1]
        yv = jnp.dot(act.astype(jnp.bfloat16), wdbf_scr[...],
                     preferred_element_type=jnp.float32)
        yi = lax.bitcast_convert_type(yv, jnp.int32)
        rb = jnp.bitwise_and(lax.shift_right_logical(yi, 16), 1) + 0x7FFF
        yr = yi + rb
        y_ref[...] = jnp.bitwise_or(
            lax.shift_right_logical(yr[:, :DH], 16),
            jnp.bitwise_and(yr[:, DH:], jnp.int32(-65536)))


def _gmm_call(texp, tact, xs, w_gate, w_up, w_down, wsorted):
    grid_spec = pltpu.PrefetchScalarGridSpec(
        num_scalar_prefetch=2,
        grid=(NT,),
        in_specs=[
            pl.BlockSpec((BM, DH),
                         lambda i, texp, tact: (jnp.where(tact[i] == 1, i, 0), 0)),
            pl.BlockSpec((1, D, F), lambda i, texp, tact: (texp[i], 0, 0)),
            pl.BlockSpec((1, D, F), lambda i, texp, tact: (texp[i], 0, 0)),
            pl.BlockSpec((1, F, D), lambda i, texp, tact: (texp[i], 0, 0)),
            pl.BlockSpec((BM, LANES),
                         lambda i, texp, tact: (jnp.where(tact[i] == 1, i, 0), 0)),
        ],
        out_specs=pl.BlockSpec((BM, DH), lambda i, texp, tact: (i, 0)),
        scratch_shapes=[
            pltpu.VMEM((D, F), jnp.bfloat16),
            pltpu.VMEM((D, F), jnp.bfloat16),
            pltpu.VMEM((F, D), jnp.bfloat16),
        ],
    )
    return pl.pallas_call(
        _gmm_body,
        grid_spec=grid_spec,
        out_shape=jax.ShapeDtypeStruct((M_PAD, DH), jnp.int32),
    )(texp, tact, xs, w_gate, w_up, w_down, wsorted)


def _dispatch_body(x_hbm, pos_hbm, w1_hbm, w2_hbm, xs_hbm, ws_hbm,
                   xbuf, w1buf, w2buf, idx0, idx1, sem):
    w = lax.axis_index("s") * 2 + lax.axis_index("c")
    base = w * TPW
    c0 = pltpu.async_copy(x_hbm.at[pl.ds(base, TPW)], xbuf, sem)
    c1 = pltpu.async_copy(pos_hbm.at[w, pl.ds(0, TPW)], idx0, sem)
    c2 = pltpu.async_copy(pos_hbm.at[NW + w, pl.ds(0, TPW)], idx1, sem)
    c3 = pltpu.async_copy(w1_hbm.at[pl.ds(base, TPW)], w1buf, sem)
    c4 = pltpu.async_copy(w2_hbm.at[pl.ds(base, TPW)], w2buf, sem)
    c0.wait(); c1.wait(); c2.wait(); c3.wait(); c4.wait()
    s0 = pltpu.async_copy(xbuf, xs_hbm.at[idx0], sem)
    s1 = pltpu.async_copy(xbuf, xs_hbm.at[idx1], sem)
    s2 = pltpu.async_copy(w1buf, ws_hbm.at[idx0], sem)
    s3 = pltpu.async_copy(w2buf, ws_hbm.at[idx1], sem)
    s0.wait(); s1.wait(); s2.wait(); s3.wait()


@functools.cache
def _dispatch_kernel():
    mesh = plsc.VectorSubcoreMesh(core_axis_name="c", subcore_axis_name="s")
    return pl.kernel(
        _dispatch_body,
        out_type=(
            jax.ShapeDtypeStruct((M_PAD, DH), jnp.int32),
            jax.ShapeDtypeStruct((M_PAD, LANES), jnp.float32),
        ),
        mesh=mesh,
        scratch_types=[
            pltpu.VMEM((TPW, DH), jnp.int32),
            pltpu.VMEM((TPW, LANES), jnp.float32),
            pltpu.VMEM((TPW, LANES), jnp.float32),
            pltpu.VMEM((TPW,), jnp.int32),
            pltpu.VMEM((TPW,), jnp.int32),
            pltpu.SemaphoreType.DMA,
        ],
    )


HT = TPW // 2


def _combine_body(y_hbm, pos_hbm, y0_hbm, y1_hbm, ybuf0, ybuf1, idx0, idx1, sem):
    w = lax.axis_index("s") * 2 + lax.axis_index("c")
    base = w * TPW
    c1 = pltpu.async_copy(pos_hbm.at[w, pl.ds(0, TPW)], idx0, sem)
    c2 = pltpu.async_copy(pos_hbm.at[NW + w, pl.ds(0, TPW)], idx1, sem)
    c1.wait(); c2.wait()
    g0 = pltpu.async_copy(y_hbm.at[idx0], ybuf0, sem)
    g1 = pltpu.async_copy(y_hbm.at[idx1], ybuf1, sem)
    g0.wait(); g1.wait()
    s0 = pltpu.async_copy(ybuf0, y0_hbm.at[pl.ds(base, TPW)], sem)
    s1 = pltpu.async_copy(ybuf1, y1_hbm.at[pl.ds(base, TPW)], sem)
    s0.wait(); s1.wait()


@functools.cache
def _combine_kernel():
    mesh = plsc.VectorSubcoreMesh(core_axis_name="c", subcore_axis_name="s")
    return pl.kernel(
        _combine_body,
        out_type=(
            jax.ShapeDtypeStruct((T, DH), jnp.int32),
            jax.ShapeDtypeStruct((T, DH), jnp.int32),
        ),
        mesh=mesh,
        scratch_types=[
            pltpu.VMEM((TPW, DH), jnp.int32),
            pltpu.VMEM((TPW, DH), jnp.int32),
            pltpu.VMEM((TPW,), jnp.int32),
            pltpu.VMEM((TPW,), jnp.int32),
            pltpu.SemaphoreType.DMA,
        ],
    )


def _final_body(y0_ref, y1_ref, o_ref):
    pk0 = y0_ref[...]
    pk1 = y1_ref[...]
    lo = (lax.bitcast_convert_type(lax.shift_left(pk0, 16), jnp.float32)
          + lax.bitcast_convert_type(lax.shift_left(pk1, 16), jnp.float32))
    hi = (lax.bitcast_convert_type(
              jnp.bitwise_and(pk0, jnp.int32(-65536)), jnp.float32)
          + lax.bitcast_convert_type(
              jnp.bitwise_and(pk1, jnp.int32(-65536)), jnp.float32))
    o_ref[...] = jnp.concatenate([lo, hi], axis=1)


def _final_call(y0, y1):
    return pl.pallas_call(
        _final_body,
        grid=(T // BM,),
        in_specs=[
            pl.BlockSpec((BM, DH), lambda i: (i, 0)),
            pl.BlockSpec((BM, DH), lambda i: (i, 0)),
        ],
        out_specs=pl.BlockSpec((BM, D), lambda i: (i, 0)),
        out_shape=jax.ShapeDtypeStruct((T, D), jnp.float32),
    )(y0, y1)


def kernel(hidden_states, gate_w, router_bias, w_gate, w_up, w_down):
    x = hidden_states.astype(jnp.float32)
    bias_2d = router_bias.reshape(1, E)

    posw, w1b, w2b, texp_b, tact_b, x_pk = _routing_call(x, gate_w, bias_2d)
    texp = texp_b[:NT, 0]
    tact = tact_b[:NT, 0]

    xs, wsorted = _dispatch_kernel()(x_pk, posw, w1b, w2b)
    y = _gmm_call(texp, tact, xs, w_gate, w_up, w_down, wsorted)
    y0, y1 = _combine_kernel()(y, posw)
    return _final_call(y0, y1)


# speedup vs baseline: 1.3089x; 1.1391x over previous
import functools

import jax
import jax.numpy as jnp
from jax import lax
from jax.experimental import pallas as pl
from jax.experimental.pallas import tpu as pltpu
from jax.experimental.pallas import tpu_sc as plsc

T = 2048
D = 1024
E = 8
K = 2
F = 512
ROUTED_SCALE = 1.0
EPS = 1e-20

LANES = 128
BM = 512
BM_LOG2 = BM.bit_length() - 1
M_PAD = T * K + E * BM
NT = M_PAD // BM
CB = 256
NB = T // CB
NW = 32
TPW = T // NW
DH = D // 2


def _routing_body(x_ref, g_ref, b_ref, posw_ref, w1_ref, w2_ref,
                  texp_ref, tact_ref, xpk_ref, c1_scr, c2_scr):
    x = x_ref[...]
    xi = lax.bitcast_convert_type(x, jnp.int32)
    rb = jnp.bitwise_and(lax.shift_right_logical(xi, 16), 1) + 0x7FFF
    xr = xi + rb
    lo = lax.shift_right_logical(xr[:, :DH], 16)
    hi = jnp.bitwise_and(xr[:, DH:], jnp.int32(-65536))
    xpk_ref[...] = jnp.bitwise_or(lo, hi)
    logits8 = lax.dot_general(x, g_ref[...], (((1,), (1,)), ((), ())),
                              preferred_element_type=jnp.float32)
    scores8 = jax.nn.sigmoid(logits8)
    lane = lax.broadcasted_iota(jnp.int32, (T, LANES), 1)
    scores = jnp.concatenate(
        [scores8, jnp.zeros((T, LANES - E), jnp.float32)], axis=1)
    choice = jnp.concatenate(
        [scores8 + b_ref[...], jnp.full((T, LANES - E), -1e30, jnp.float32)],
        axis=1)
    m1 = jnp.max(choice, axis=1, keepdims=True)
    i1 = jnp.min(jnp.where(choice >= m1, lane, LANES), axis=1, keepdims=True)
    sel1 = lane == i1
    w1 = jnp.sum(jnp.where(sel1, scores, 0.0), axis=1, keepdims=True)
    choice2 = jnp.where(sel1, -1e30, choice)
    m2 = jnp.max(choice2, axis=1, keepdims=True)
    i2 = jnp.min(jnp.where(choice2 >= m2, lane, LANES), axis=1, keepdims=True)
    sel2 = lane == i2
    w2 = jnp.sum(jnp.where(sel2, scores, 0.0), axis=1, keepdims=True)
    den = w1 + w2 + EPS
    w1 = w1 / den * ROUTED_SCALE
    w2 = w2 / den * ROUTED_SCALE
    w1_ref[...] = jnp.broadcast_to(w1, (T, LANES))
    w2_ref[...] = jnp.broadcast_to(w2, (T, LANES))

    oh1 = sel1.astype(jnp.float32)
    oh2 = sel2.astype(jnp.float32)
    c1_scr[...] = oh1
    c2_scr[...] = oh2
    ltexc = (lax.broadcasted_iota(jnp.int32, (CB, CB), 0)
             > lax.broadcasted_iota(jnp.int32, (CB, CB), 1)).astype(jnp.float32)

    def blk(b, carry):
        car1, car2 = carry
        s = pl.ds(b * CB, CB)
        blk1 = c1_scr[s, :]
        blk2 = c2_scr[s, :]
        c1_scr[s, :] = lax.dot_general(
            ltexc, blk1, (((1,), (0,)), ((), ())),
            preferred_element_type=jnp.float32) + car1
        c2_scr[s, :] = lax.dot_general(
            ltexc, blk2, (((1,), (0,)), ((), ())),
            preferred_element_type=jnp.float32) + car2
        return (car1 + jnp.sum(blk1, axis=0, keepdims=True),
                car2 + jnp.sum(blk2, axis=0, keepdims=True))

    zero = jnp.zeros((1, LANES), jnp.float32)
    tot1, tot2 = lax.fori_loop(0, NB, blk, (zero, zero))

    counts = (tot1 + tot2).astype(jnp.int32)
    padded = lax.shift_left(
        lax.shift_right_logical(counts + (BM - 1), BM_LOG2), BM_LOG2)
    lte = (lax.broadcasted_iota(jnp.int32, (LANES, LANES), 0)
           < lax.broadcasted_iota(jnp.int32, (LANES, LANES), 1)).astype(jnp.float32)
    seg = lax.dot_general(padded.astype(jnp.float32), lte,
                          (((1,), (0,)), ((), ())),
                          preferred_element_type=jnp.float32)
    c1v = c1_scr[...]
    c2v = c2_scr[...]
    pos1 = jnp.sum((seg + c1v) * oh1, axis=1, keepdims=True)
    pos2 = jnp.sum((seg + tot1 + c2v) * oh2, axis=1, keepdims=True)
    tok0 = lax.broadcasted_iota(jnp.int32, (T, LANES), 0)
    eqj = (lane == jnp.bitwise_and(tok0, TPW - 1)).astype(jnp.float32)
    wsel = (lax.broadcasted_iota(jnp.int32, (NW, T), 0)
            == lax.shift_right_logical(
                lax.broadcasted_iota(jnp.int32, (NW, T), 1), 6)
            ).astype(jnp.float32)
    cdims = (((1,), (0,)), ((), ()))

    def relayout(posv):
        p_hi = jnp.floor(posv * (1.0 / 32.0))
        p_lo = posv - 32.0 * p_hi
        hi = lax.dot_general(wsel, p_hi * eqj, cdims,
                             preferred_element_type=jnp.float32)
        lo = lax.dot_general(wsel, p_lo * eqj, cdims,
                             preferred_element_type=jnp.float32)
        return 32.0 * hi + lo

    posw_ref[...] = jnp.concatenate(
        [relayout(pos1), relayout(pos2)], axis=0).astype(jnp.int32)

    segi = seg.astype(jnp.int32)
    tstart = lax.broadcasted_iota(jnp.int32, (32, LANES), 0) * BM
    act2 = (segi <= tstart) & (tstart < segi + counts)
    lane2 = lax.broadcasted_iota(jnp.int32, (32, LANES), 1)
    tact = jnp.sum(act2.astype(jnp.int32), axis=1, keepdims=True)
    texp = jnp.where(tact == 1,
                     jnp.sum(jnp.where(act2, lane2, 0), axis=1, keepdims=True),
                     E - 1)
    texp_ref[...] = jnp.broadcast_to(texp, (32, LANES))
    tact_ref[...] = jnp.broadcast_to(tact, (32, LANES))


def _routing_call(x, gate_w, bias_2d):
    return pl.pallas_call(
        _routing_body,
        out_shape=(
            jax.ShapeDtypeStruct((2 * NW, LANES), jnp.int32),
            jax.ShapeDtypeStruct((T, LANES), jnp.float32),
            jax.ShapeDtypeStruct((T, LANES), jnp.float32),
            jax.ShapeDtypeStruct((32, LANES), jnp.int32),
            jax.ShapeDtypeStruct((32, LANES), jnp.int32),
            jax.ShapeDtypeStruct((T, DH), jnp.int32),
        ),
        scratch_shapes=[
            pltpu.VMEM((T, LANES), jnp.float32),
            pltpu.VMEM((T, LANES), jnp.float32),
        ],
    )(x, gate_w, bias_2d)


def _gmm_body(texp_ref, tact_ref, xs_ref, wg_ref, wu_ref, wd_ref, ws_ref, y_ref,
              wgbf_scr, wubf_scr, wdbf_scr):
    i = pl.program_id(0)
    changed = jnp.logical_or(
        i == 0, texp_ref[i] != texp_ref[jnp.maximum(i - 1, 0)])

    @pl.when(jnp.logical_and(tact_ref[i] == 1, changed))
    def _():
        wgbf_scr[...] = wg_ref[0].astype(jnp.bfloat16)
        wubf_scr[...] = wu_ref[0].astype(jnp.bfloat16)
        wdbf_scr[...] = wd_ref[0].astype(jnp.bfloat16)

    @pl.when(tact_ref[i] == 1)
    def _():
        pk = xs_ref[...]
        xlo = lax.bitcast_convert_type(
            lax.shift_left(pk, 16), jnp.float32).astype(jnp.bfloat16)
        xhi = lax.bitcast_convert_type(
            jnp.bitwise_and(pk, jnp.int32(-65536)), jnp.float32).astype(jnp.bfloat16)
        xt = jnp.concatenate([xlo, xhi], axis=1)
        g = jnp.dot(xt, wgbf_scr[...], preferred_element_type=jnp.float32)
        u = jnp.dot(xt, wubf_scr[...], preferred_element_type=jnp.float32)
        act = g * jax.nn.sigmoid(g) * u
        act = act * ws_ref[:, :1]
        yv = jnp.dot(act.astype(jnp.bfloat16), wdbf_scr[...],
                     preferred_element_type=jnp.float32)
        yi = lax.bitcast_convert_type(yv, jnp.int32)
        rb = jnp.bitwise_and(lax.shift_right_logical(yi, 16), 1) + 0x7FFF
        yr = yi + rb
        y_ref[...] = jnp.bitwise_or(
            lax.shift_right_logical(yr[:, :DH], 16),
            jnp.bitwise_and(yr[:, DH:], jnp.int32(-65536)))


def _gmm_call(texp, tact, xs, w_gate, w_up, w_down, wsorted):
    grid_spec = pltpu.PrefetchScalarGridSpec(
        num_scalar_prefetch=2,
        grid=(NT,),
        in_specs=[
            pl.BlockSpec((BM, DH),
                         lambda i, texp, tact: (jnp.where(tact[i] == 1, i, 0), 0)),
            pl.BlockSpec((1, D, F), lambda i, texp, tact: (texp[i], 0, 0)),
            pl.BlockSpec((1, D, F), lambda i, texp, tact: (texp[i], 0, 0)),
            pl.BlockSpec((1, F, D), lambda i, texp, tact: (texp[i], 0, 0)),
            pl.BlockSpec((BM, LANES),
                         lambda i, texp, tact: (jnp.where(tact[i] == 1, i, 0), 0)),
        ],
        out_specs=pl.BlockSpec((BM, DH), lambda i, texp, tact: (i, 0)),
        scratch_shapes=[
            pltpu.VMEM((D, F), jnp.bfloat16),
            pltpu.VMEM((D, F), jnp.bfloat16),
            pltpu.VMEM((F, D), jnp.bfloat16),
        ],
    )
    return pl.pallas_call(
        _gmm_body,
        grid_spec=grid_spec,
        out_shape=jax.ShapeDtypeStruct((M_PAD, DH), jnp.int32),
    )(texp, tact, xs, w_gate, w_up, w_down, wsorted)


def _dispatch_body(x_hbm, pos_hbm, w1_hbm, w2_hbm, xs_hbm, ws_hbm,
                   xbuf, w1buf, w2buf, idx0, idx1, sem):
    w = lax.axis_index("s") * 2 + lax.axis_index("c")
    base = w * TPW
    c0 = pltpu.async_copy(x_hbm.at[pl.ds(base, TPW)], xbuf, sem)
    c1 = pltpu.async_copy(pos_hbm.at[w, pl.ds(0, TPW)], idx0, sem)
    c2 = pltpu.async_copy(pos_hbm.at[NW + w, pl.ds(0, TPW)], idx1, sem)
    c3 = pltpu.async_copy(w1_hbm.at[pl.ds(base, TPW)], w1buf, sem)
    c4 = pltpu.async_copy(w2_hbm.at[pl.ds(base, TPW)], w2buf, sem)
    c0.wait(); c1.wait(); c2.wait(); c3.wait(); c4.wait()
    s0 = pltpu.async_copy(xbuf, xs_hbm.at[idx0], sem)
    s1 = pltpu.async_copy(xbuf, xs_hbm.at[idx1], sem)
    s2 = pltpu.async_copy(w1buf, ws_hbm.at[idx0], sem)
    s3 = pltpu.async_copy(w2buf, ws_hbm.at[idx1], sem)
    s0.wait(); s1.wait(); s2.wait(); s3.wait()


@functools.cache
def _dispatch_kernel():
    mesh = plsc.VectorSubcoreMesh(core_axis_name="c", subcore_axis_name="s")
    return pl.kernel(
        _dispatch_body,
        out_type=(
            jax.ShapeDtypeStruct((M_PAD, DH), jnp.int32),
            jax.ShapeDtypeStruct((M_PAD, LANES), jnp.float32),
        ),
        mesh=mesh,
        scratch_types=[
            pltpu.VMEM((TPW, DH), jnp.int32),
            pltpu.VMEM((TPW, LANES), jnp.float32),
            pltpu.VMEM((TPW, LANES), jnp.float32),
            pltpu.VMEM((TPW,), jnp.int32),
            pltpu.VMEM((TPW,), jnp.int32),
            pltpu.SemaphoreType.DMA,
        ],
    )


HT = TPW // 2


def _combine_body(y_hbm, pos_hbm, y0_hbm, y1_hbm, ybuf0, ybuf1, idx0, idx1, sem):
    w = lax.axis_index("s") * 2 + lax.axis_index("c")
    base = w * TPW
    c1 = pltpu.async_copy(pos_hbm.at[w, pl.ds(0, TPW)], idx0, sem)
    c2 = pltpu.async_copy(pos_hbm.at[NW + w, pl.ds(0, TPW)], idx1, sem)
    c1.wait(); c2.wait()
    g0 = pltpu.async_copy(y_hbm.at[idx0], ybuf0, sem)
    g1 = pltpu.async_copy(y_hbm.at[idx1], ybuf1, sem)
    g0.wait(); g1.wait()
    s0 = pltpu.async_copy(ybuf0, y0_hbm.at[pl.ds(base, TPW)], sem)
    s1 = pltpu.async_copy(ybuf1, y1_hbm.at[pl.ds(base, TPW)], sem)
    s0.wait(); s1.wait()


@functools.cache
def _combine_kernel():
    mesh = plsc.VectorSubcoreMesh(core_axis_name="c", subcore_axis_name="s")
    return pl.kernel(
        _combine_body,
        out_type=(
            jax.ShapeDtypeStruct((T, DH), jnp.int32),
            jax.ShapeDtypeStruct((T, DH), jnp.int32),
        ),
        mesh=mesh,
        scratch_types=[
            pltpu.VMEM((TPW, DH), jnp.int32),
            pltpu.VMEM((TPW, DH), jnp.int32),
            pltpu.VMEM((TPW,), jnp.int32),
            pltpu.VMEM((TPW,), jnp.int32),
            pltpu.SemaphoreType.DMA,
        ],
    )


def _final_body(y0_ref, y1_ref, o_ref):
    pk0 = y0_ref[...]
    pk1 = y1_ref[...]
    lo = (lax.bitcast_convert_type(lax.shift_left(pk0, 16), jnp.float32)
          + lax.bitcast_convert_type(lax.shift_left(pk1, 16), jnp.float32))
    hi = (lax.bitcast_convert_type(
              jnp.bitwise_and(pk0, jnp.int32(-65536)), jnp.float32)
          + lax.bitcast_convert_type(
              jnp.bitwise_and(pk1, jnp.int32(-65536)), jnp.float32))
    o_ref[...] = jnp.concatenate([lo, hi], axis=1)


def _final_call(y0, y1):
    return pl.pallas_call(
        _final_body,
        grid=(T // BM,),
        in_specs=[
            pl.BlockSpec((BM, DH), lambda i: (i, 0)),
            pl.BlockSpec((BM, DH), lambda i: (i, 0)),
        ],
        out_specs=pl.BlockSpec((BM, D), lambda i: (i, 0)),
        out_shape=jax.ShapeDtypeStruct((T, D), jnp.float32),
    )(y0, y1)


def kernel(hidden_states, gate_w, router_bias, w_gate, w_up, w_down):
    x = hidden_states.astype(jnp.float32)
    bias_2d = router_bias.reshape(1, E)

    posw, w1b, w2b, texp_b, tact_b, x_pk = _routing_call(x, gate_w, bias_2d)
    texp = texp_b[:NT, 0]
    tact = tact_b[:NT, 0]

    xs, wsorted = _dispatch_kernel()(x_pk, posw, w1b, w2b)
    y = _gmm_call(texp, tact, xs, w_gate, w_up, w_down, wsorted)
    y0, y1 = _combine_kernel()(y, posw)
    return _final_call(y0, y1)


# speedup vs baseline: 1.3222x; 1.0101x over previous
import functools

import jax
import jax.numpy as jnp
from jax import lax
from jax.experimental import pallas as pl
from jax.experimental.pallas import tpu as pltpu
from jax.experimental.pallas import tpu_sc as plsc

T = 2048
D = 1024
E = 8
K = 2
F = 512
ROUTED_SCALE = 1.0
EPS = 1e-20

LANES = 128
BM = 512
BM_LOG2 = BM.bit_length() - 1
M_PAD = T * K + E * BM
NT = M_PAD // BM
CB = 256
NB = T // CB
NW = 32
TPW = T // NW
DH = D // 2


def _routing_body(x_ref, g_ref, b_ref, posw_ref, w1_ref, w2_ref,
                  texp_ref, tact_ref, xpk_ref, c1_scr, c2_scr):
    x = x_ref[...]
    xi = lax.bitcast_convert_type(x, jnp.int32)
    rb = jnp.bitwise_and(lax.shift_right_logical(xi, 16), 1) + 0x7FFF
    xr = xi + rb
    lo = lax.shift_right_logical(xr[:, :DH], 16)
    hi = jnp.bitwise_and(xr[:, DH:], jnp.int32(-65536))
    xpk_ref[...] = jnp.bitwise_or(lo, hi)
    logits8 = lax.dot_general(x, g_ref[...], (((1,), (1,)), ((), ())),
                              preferred_element_type=jnp.float32)
    scores8 = jax.nn.sigmoid(logits8)
    lane = lax.broadcasted_iota(jnp.int32, (T, LANES), 1)
    scores = jnp.concatenate(
        [scores8, jnp.zeros((T, LANES - E), jnp.float32)], axis=1)
    choice = jnp.concatenate(
        [scores8 + b_ref[...], jnp.full((T, LANES - E), -1e30, jnp.float32)],
        axis=1)
    m1 = jnp.max(choice, axis=1, keepdims=True)
    i1 = jnp.min(jnp.where(choice >= m1, lane, LANES), axis=1, keepdims=True)
    sel1 = lane == i1
    w1 = jnp.sum(jnp.where(sel1, scores, 0.0), axis=1, keepdims=True)
    choice2 = jnp.where(sel1, -1e30, choice)
    m2 = jnp.max(choice2, axis=1, keepdims=True)
    i2 = jnp.min(jnp.where(choice2 >= m2, lane, LANES), axis=1, keepdims=True)
    sel2 = lane == i2
    w2 = jnp.sum(jnp.where(sel2, scores, 0.0), axis=1, keepdims=True)
    den = w1 + w2 + EPS
    w1 = w1 / den * ROUTED_SCALE
    w2 = w2 / den * ROUTED_SCALE
    w1_ref[...] = jnp.broadcast_to(w1, (T, LANES))
    w2_ref[...] = jnp.broadcast_to(w2, (T, LANES))

    oh1 = sel1.astype(jnp.float32)
    oh2 = sel2.astype(jnp.float32)
    c1_scr[...] = oh1
    c2_scr[...] = oh2
    ltexc = (lax.broadcasted_iota(jnp.int32, (CB, CB), 0)
             > lax.broadcasted_iota(jnp.int32, (CB, CB), 1)).astype(jnp.float32)

    def blk(b, carry):
        car1, car2 = carry
        s = pl.ds(b * CB, CB)
        blk1 = c1_scr[s, :]
        blk2 = c2_scr[s, :]
        c1_scr[s, :] = lax.dot_general(
            ltexc, blk1, (((1,), (0,)), ((), ())),
            preferred_element_type=jnp.float32) + car1
        c2_scr[s, :] = lax.dot_general(
            ltexc, blk2, (((1,), (0,)), ((), ())),
            preferred_element_type=jnp.float32) + car2
        return (car1 + jnp.sum(blk1, axis=0, keepdims=True),
                car2 + jnp.sum(blk2, axis=0, keepdims=True))

    zero = jnp.zeros((1, LANES), jnp.float32)
    tot1, tot2 = lax.fori_loop(0, NB, blk, (zero, zero))

    counts = (tot1 + tot2).astype(jnp.int32)
    padded = lax.shift_left(
        lax.shift_right_logical(counts + (BM - 1), BM_LOG2), BM_LOG2)
    lte = (lax.broadcasted_iota(jnp.int32, (LANES, LANES), 0)
           < lax.broadcasted_iota(jnp.int32, (LANES, LANES), 1)).astype(jnp.float32)
    seg = lax.dot_general(padded.astype(jnp.float32), lte,
                          (((1,), (0,)), ((), ())),
                          preferred_element_type=jnp.float32)
    c1v = c1_scr[...]
    c2v = c2_scr[...]
    pos1 = jnp.sum((seg + c1v) * oh1, axis=1, keepdims=True)
    pos2 = jnp.sum((seg + tot1 + c2v) * oh2, axis=1, keepdims=True)
    tok0 = lax.broadcasted_iota(jnp.int32, (T, LANES), 0)
    eqj = (lane == jnp.bitwise_and(tok0, TPW - 1)).astype(jnp.float32)
    wsel = (lax.broadcasted_iota(jnp.int32, (NW, T), 0)
            == lax.shift_right_logical(
                lax.broadcasted_iota(jnp.int32, (NW, T), 1), 6)
            ).astype(jnp.float32)
    cdims = (((1,), (0,)), ((), ()))

    def relayout(posv):
        p_hi = jnp.floor(posv * (1.0 / 32.0))
        p_lo = posv - 32.0 * p_hi
        hi = lax.dot_general(wsel, p_hi * eqj, cdims,
                             preferred_element_type=jnp.float32)
        lo = lax.dot_general(wsel, p_lo * eqj, cdims,
                             preferred_element_type=jnp.float32)
        return 32.0 * hi + lo

    posw_ref[...] = jnp.concatenate(
        [relayout(pos1), relayout(pos2)], axis=0).astype(jnp.int32)

    segi = seg.astype(jnp.int32)
    tstart = lax.broadcasted_iota(jnp.int32, (32, LANES), 0) * BM
    act2 = (segi <= tstart) & (tstart < segi + counts)
    lane2 = lax.broadcasted_iota(jnp.int32, (32, LANES), 1)
    tact = jnp.sum(act2.astype(jnp.int32), axis=1, keepdims=True)
    texp = jnp.where(tact == 1,
                     jnp.sum(jnp.where(act2, lane2, 0), axis=1, keepdims=True),
                     E - 1)
    texp_ref[...] = jnp.broadcast_to(texp, (32, LANES))
    tact_ref[...] = jnp.broadcast_to(tact, (32, LANES))


def _routing_call(x, gate_w, bias_2d):
    return pl.pallas_call(
        _routing_body,
        out_shape=(
            jax.ShapeDtypeStruct((2 * NW, LANES), jnp.int32),
            jax.ShapeDtypeStruct((T, LANES), jnp.float32),
            jax.ShapeDtypeStruct((T, LANES), jnp.float32),
            jax.ShapeDtypeStruct((32, LANES), jnp.int32),
            jax.ShapeDtypeStruct((32, LANES), jnp.int32),
            jax.ShapeDtypeStruct((T, DH), jnp.int32),
        ),
        scratch_shapes=[
            pltpu.VMEM((T, LANES), jnp.float32),
            pltpu.VMEM((T, LANES), jnp.float32),
        ],
    )(x, gate_w, bias_2d)


def _gmm_body(texp_ref, tact_ref, xs_ref, wg_ref, wu_ref, wd_ref, y_ref,
              wgbf_scr, wubf_scr, wdbf_scr):
    i = pl.program_id(0)
    changed = jnp.logical_or(
        i == 0, texp_ref[i] != texp_ref[jnp.maximum(i - 1, 0)])

    @pl.when(jnp.logical_and(tact_ref[i] == 1, changed))
    def _():
        wgbf_scr[...] = wg_ref[0].astype(jnp.bfloat16)
        wubf_scr[...] = wu_ref[0].astype(jnp.bfloat16)
        wdbf_scr[...] = wd_ref[0].astype(jnp.bfloat16)

    @pl.when(tact_ref[i] == 1)
    def _():
        pk = xs_ref[...]
        xlo = lax.bitcast_convert_type(
            lax.shift_left(pk, 16), jnp.float32).astype(jnp.bfloat16)
        xhi = lax.bitcast_convert_type(
            jnp.bitwise_and(pk, jnp.int32(-65536)), jnp.float32).astype(jnp.bfloat16)
        xt = jnp.concatenate([xlo, xhi], axis=1)
        g = jnp.dot(xt, wgbf_scr[...], preferred_element_type=jnp.float32)
        u = jnp.dot(xt, wubf_scr[...], preferred_element_type=jnp.float32)
        act = g * jax.nn.sigmoid(g) * u
        yv = jnp.dot(act.astype(jnp.bfloat16), wdbf_scr[...],
                     preferred_element_type=jnp.float32)
        yi = lax.bitcast_convert_type(yv, jnp.int32)
        rb = jnp.bitwise_and(lax.shift_right_logical(yi, 16), 1) + 0x7FFF
        yr = yi + rb
        y_ref[...] = jnp.bitwise_or(
            lax.shift_right_logical(yr[:, :DH], 16),
            jnp.bitwise_and(yr[:, DH:], jnp.int32(-65536)))


def _gmm_call(texp, tact, xs, w_gate, w_up, w_down):
    grid_spec = pltpu.PrefetchScalarGridSpec(
        num_scalar_prefetch=2,
        grid=(NT,),
        in_specs=[
            pl.BlockSpec((BM, DH),
                         lambda i, texp, tact: (jnp.where(tact[i] == 1, i, 0), 0)),
            pl.BlockSpec((1, D, F), lambda i, texp, tact: (texp[i], 0, 0)),
            pl.BlockSpec((1, D, F), lambda i, texp, tact: (texp[i], 0, 0)),
            pl.BlockSpec((1, F, D), lambda i, texp, tact: (texp[i], 0, 0)),
        ],
        out_specs=pl.BlockSpec((BM, DH), lambda i, texp, tact: (i, 0)),
        scratch_shapes=[
            pltpu.VMEM((D, F), jnp.bfloat16),
            pltpu.VMEM((D, F), jnp.bfloat16),
            pltpu.VMEM((F, D), jnp.bfloat16),
        ],
    )
    return pl.pallas_call(
        _gmm_body,
        grid_spec=grid_spec,
        out_shape=jax.ShapeDtypeStruct((M_PAD, DH), jnp.int32),
    )(texp, tact, xs, w_gate, w_up, w_down)


def _dispatch_body(x_hbm, pos_hbm, xs_hbm, xbuf, idx0, idx1, sem):
    w = lax.axis_index("s") * 2 + lax.axis_index("c")
    base = w * TPW
    c0 = pltpu.async_copy(x_hbm.at[pl.ds(base, TPW)], xbuf, sem)
    c1 = pltpu.async_copy(pos_hbm.at[w, pl.ds(0, TPW)], idx0, sem)
    c2 = pltpu.async_copy(pos_hbm.at[NW + w, pl.ds(0, TPW)], idx1, sem)
    c0.wait(); c1.wait(); c2.wait()
    s0 = pltpu.async_copy(xbuf, xs_hbm.at[idx0], sem)
    s1 = pltpu.async_copy(xbuf, xs_hbm.at[idx1], sem)
    s0.wait(); s1.wait()


@functools.cache
def _dispatch_kernel():
    mesh = plsc.VectorSubcoreMesh(core_axis_name="c", subcore_axis_name="s")
    return pl.kernel(
        _dispatch_body,
        out_type=jax.ShapeDtypeStruct((M_PAD, DH), jnp.int32),
        mesh=mesh,
        scratch_types=[
            pltpu.VMEM((TPW, DH), jnp.int32),
            pltpu.VMEM((TPW,), jnp.int32),
            pltpu.VMEM((TPW,), jnp.int32),
            pltpu.SemaphoreType.DMA,
        ],
    )


HT = TPW // 2


def _combine_body(y_hbm, pos_hbm, y0_hbm, y1_hbm, ybuf0, ybuf1, idx0, idx1, sem):
    w = lax.axis_index("s") * 2 + lax.axis_index("c")
    base = w * TPW
    c1 = pltpu.async_copy(pos_hbm.at[w, pl.ds(0, TPW)], idx0, sem)
    c2 = pltpu.async_copy(pos_hbm.at[NW + w, pl.ds(0, TPW)], idx1, sem)
    c1.wait(); c2.wait()
    g0 = pltpu.async_copy(y_hbm.at[idx0], ybuf0, sem)
    g1 = pltpu.async_copy(y_hbm.at[idx1], ybuf1, sem)
    g0.wait(); g1.wait()
    s0 = pltpu.async_copy(ybuf0, y0_hbm.at[pl.ds(base, TPW)], sem)
    s1 = pltpu.async_copy(ybuf1, y1_hbm.at[pl.ds(base, TPW)], sem)
    s0.wait(); s1.wait()


@functools.cache
def _combine_kernel():
    mesh = plsc.VectorSubcoreMesh(core_axis_name="c", subcore_axis_name="s")
    return pl.kernel(
        _combine_body,
        out_type=(
            jax.ShapeDtypeStruct((T, DH), jnp.int32),
            jax.ShapeDtypeStruct((T, DH), jnp.int32),
        ),
        mesh=mesh,
        scratch_types=[
            pltpu.VMEM((TPW, DH), jnp.int32),
            pltpu.VMEM((TPW, DH), jnp.int32),
            pltpu.VMEM((TPW,), jnp.int32),
            pltpu.VMEM((TPW,), jnp.int32),
            pltpu.SemaphoreType.DMA,
        ],
    )


def _final_body(y0_ref, y1_ref, w1_ref, w2_ref, o_ref):
    pk0 = y0_ref[...]
    pk1 = y1_ref[...]
    w1 = w1_ref[:, :1]
    w2 = w2_ref[:, :1]
    lo = (w1 * lax.bitcast_convert_type(lax.shift_left(pk0, 16), jnp.float32)
          + w2 * lax.bitcast_convert_type(lax.shift_left(pk1, 16), jnp.float32))
    hi = (w1 * lax.bitcast_convert_type(
              jnp.bitwise_and(pk0, jnp.int32(-65536)), jnp.float32)
          + w2 * lax.bitcast_convert_type(
              jnp.bitwise_and(pk1, jnp.int32(-65536)), jnp.float32))
    o_ref[...] = jnp.concatenate([lo, hi], axis=1)


def _final_call(y0, y1, w1b, w2b):
    return pl.pallas_call(
        _final_body,
        grid=(T // BM,),
        in_specs=[
            pl.BlockSpec((BM, DH), lambda i: (i, 0)),
            pl.BlockSpec((BM, DH), lambda i: (i, 0)),
            pl.BlockSpec((BM, LANES), lambda i: (i, 0)),
            pl.BlockSpec((BM, LANES), lambda i: (i, 0)),
        ],
        out_specs=pl.BlockSpec((BM, D), lambda i: (i, 0)),
        out_shape=jax.ShapeDtypeStruct((T, D), jnp.float32),
    )(y0, y1, w1b, w2b)


def kernel(hidden_states, gate_w, router_bias, w_gate, w_up, w_down):
    x = hidden_states.astype(jnp.float32)
    bias_2d = router_bias.reshape(1, E)

    posw, w1b, w2b, texp_b, tact_b, x_pk = _routing_call(x, gate_w, bias_2d)
    texp = texp_b[:NT, 0]
    tact = tact_b[:NT, 0]

    xs = _dispatch_kernel()(x_pk, posw)
    y = _gmm_call(texp, tact, xs, w_gate, w_up, w_down)
    y0, y1 = _combine_kernel()(y, posw)
    return _final_call(y0, y1, w1b, w2b)
